# baseline (speedup 1.0000x reference)
import jax
import jax.numpy as jnp
from jax import lax
from jax.experimental import pallas as pl
from jax.experimental.pallas import tpu as pltpu
from jax.experimental.pallas import tpu_sc as plsc

F = 32
B = 262144
GD = 96
PD = 384
NC, NS, L = 2, 16, 16
NW = NC * NS
SPT = B // NW
C = 64
NCHUNK = SPT // C
NG = C // L

def _vgather(v, idx):
    return lax.gather(
        v, idx[:, None],
        dimension_numbers=lax.GatherDimensionNumbers(
            offset_dims=(), collapsed_slice_dims=(0,), start_index_map=(0,)),
        slice_sizes=(1,),
        mode=lax.GatherScatterMode.PROMISE_IN_BOUNDS)


G_OFF = (0, 1, GD, GD + 1, GD * GD, GD * GD + 1, GD * GD + GD, GD * GD + GD + 1)
P_OFF = (0, 1, PD, PD + 1)


def _sc_body(grid_t, p0_t, p1_t, p2_t, xs, ys, zs, out,
             xv, yv, zv,
             idxg0, idxp00, idxp10, idxp20, rg0, rp00, rp10, rp20, fr0, ob0,
             idxg1, idxp01, idxp11, idxp21, rg1, rp01, rp11, rp21, fr1, ob1,
             gsem0, gsem1, osem0, osem1):
    wid = lax.axis_index("s") * NC + lax.axis_index("c")
    tbase = wid * SPT

    buf0 = (idxg0, idxp00, idxp10, idxp20, rg0, rp00, rp10, rp20, fr0, ob0,
            gsem0, osem0)
    buf1 = (idxg1, idxp01, idxp11, idxp21, rg1, rp01, rp11, rp21, fr1, ob1,
            gsem1, osem1)

    pltpu.sync_copy(xs.at[pl.ds(tbase, SPT)], xv)
    pltpu.sync_copy(ys.at[pl.ds(tbase, SPT)], yv)
    pltpu.sync_copy(zs.at[pl.ds(tbase, SPT)], zv)

    lane = lax.iota(jnp.int32, L)
    ilo0 = lax.shift_right_logical(lane, 1)
    ilo1 = ilo0 + (L // 2)
    sel_even = (lane & 1) == 0
    g_scale = jnp.float32(0.5 * (GD - 1))
    p_scale = jnp.float32(0.5 * (PD - 1))

    def stage_a(kc, buf):
        idxg, idxp0, idxp1, idxp2, _, _, _, _, fr = buf[:9]

        def body(g, kcc):
            s0 = pl.multiple_of(kcc * C + g * L, L)
            g16 = pl.multiple_of(g * L, L)
            cx = xv[pl.ds(s0, L)] + 1.0
            cy = yv[pl.ds(s0, L)] + 1.0
            cz = zv[pl.ds(s0, L)] + 1.0

            def split(t, scale, hi):
                ti = t * scale
                i0 = jnp.minimum(ti.astype(jnp.int32), hi)
                frac = ti - i0.astype(jnp.float32)
                return i0, frac

            x0, fx = split(cx, g_scale, GD - 2)
            y0, fy = split(cy, g_scale, GD - 2)
            z0, fz = split(cz, g_scale, GD - 2)
            xp, fxp = split(cx, p_scale, PD - 2)
            yp, fyp = split(cy, p_scale, PD - 2)
            zp, fzp = split(cz, p_scale, PD - 2)

            r0 = (z0 * GD + y0) * GD + x0
            for c, off in enumerate(G_OFF):
                idxg[pl.ds(c * C + g16, L)] = r0 + off
            rp0 = yp * PD + xp
            rp1 = zp * PD + xp
            rp2 = zp * PD + yp
            for c, off in enumerate(P_OFF):
                idxp0[pl.ds(c * C + g16, L)] = rp0 + off
                idxp1[pl.ds(c * C + g16, L)] = rp1 + off
                idxp2[pl.ds(c * C + g16, L)] = rp2 + off
            for j, v in enumerate((fx, fy, fz, fxp, fyp, fzp)):
                fr[j, pl.ds(g16, L)] = v
            return kcc

        lax.fori_loop(0, NG, body, kc)

    def gather_pairs(buf):
        idxg, idxp0, idxp1, idxp2, rg, rp0, rp1, rp2 = buf[:8]
        gsem = buf[10]
        return ((grid_t.at[idxg], rg, gsem),
                (p0_t.at[idxp0], rp0, gsem),
                (p1_t.at[idxp1], rp1, gsem),
                (p2_t.at[idxp2], rp2, gsem))

    def fire(buf):
        for src, dst, sem in gather_pairs(buf):
            pltpu.async_copy(src, dst, sem)

    def wait_gathers(buf):
        for src, dst, sem in gather_pairs(buf):
            pltpu.make_async_copy(src, dst, sem).wait()

    def stage_c(kc, buf):
        rg, rp0, rp1, rp2, fr, ob = buf[4:10]
        prefs = (rp0, rp1, rp2)

        def body(g, kcc):
            g16 = pl.multiple_of(g * L, L)
            fxv = fr[0, pl.ds(g16, L)]
            fyv = fr[1, pl.ds(g16, L)]
            fzv = fr[2, pl.ds(g16, L)]
            fxpv = fr[3, pl.ds(g16, L)]
            fypv = fr[4, pl.ds(g16, L)]
            fzpv = fr[5, pl.ds(g16, L)]
            for sl in range(L):
                s = g16 + sl
                fx = jnp.full((L,), fxv[sl], jnp.float32)
                fy = jnp.full((L,), fyv[sl], jnp.float32)
                fz = jnp.full((L,), fzv[sl], jnp.float32)
                fxp = jnp.full((L,), fxpv[sl], jnp.float32)
                fyp = jnp.full((L,), fypv[sl], jnp.float32)
                fzp = jnp.full((L,), fzpv[sl], jnp.float32)
                wx0, wx1 = 1.0 - fx, fx
                wy0, wy1 = 1.0 - fy, fy
                wz0, wz1 = 1.0 - fz, fz
                a00 = wz0 * wy0
                a01 = wz0 * wy1
                a10 = wz1 * wy0
                a11 = wz1 * wy1
                wg = (a00 * wx0, a00 * wx1, a01 * wx0, a01 * wx1,
                      a10 * wx0, a10 * wx1, a11 * wx0, a11 * wx1)
                wxp0, wxp1 = 1.0 - fxp, fxp
                wyp0, wyp1 = 1.0 - fyp, fyp
                wzp0, wzp1 = 1.0 - fzp, fzp
                wp = ((wyp0 * wxp0, wyp0 * wxp1, wyp1 * wxp0, wyp1 * wxp1),
                      (wzp0 * wxp0, wzp0 * wxp1, wzp1 * wxp0, wzp1 * wxp1),
                      (wzp0 * wyp0, wzp0 * wyp1, wzp1 * wyp0, wzp1 * wyp1))

                def halves(w):
                    lo = lax.bitcast_convert_type(lax.shift_left(w, 16), jnp.float32)
                    hi = lax.bitcast_convert_type(
                        lax.bitwise_and(w, jnp.int32(-65536)), jnp.float32)
                    return lo, hi

                acc = [None, None]
                for c in range(8):
                    v0, v1 = halves(rg[c * C + s, :])
                    for h, v in enumerate((v0, v1)):
                        t = wg[c] * v
                        acc[h] = t if c == 0 else acc[h] + t
                for p in range(3):
                    pacc = [None, None]
                    for c in range(4):
                        v0, v1 = halves(prefs[p][c * C + s, :])
                        for h, v in enumerate((v0, v1)):
                            t = wp[p][c] * v
                            pacc[h] = t if c == 0 else pacc[h] + t
                    for h in range(2):
                        acc[h] = acc[h] * (1.0 + pacc[h])
                e, o = acc
                lo0 = _vgather(e, ilo0)
                lo1 = _vgather(o, ilo0)
                hi0 = _vgather(e, ilo1)
                hi1 = _vgather(o, ilo1)
                ob[s, pl.ds(0, L)] = jnp.where(sel_even, lo0, lo1)
                ob[s, pl.ds(L, L)] = jnp.where(sel_even, hi0, hi1)
            return kcc

        lax.fori_loop(0, NG, body, kc)

    def fire_out(kc, buf):
        ob, _, osem = buf[9:12]
        pltpu.async_copy(ob, out.at[pl.ds(tbase + kc * C, C)], osem)

    def wait_out(buf):
        ob, _, osem = buf[9:12]
        pltpu.make_async_copy(ob, out.at[pl.ds(tbase, C)], osem).wait()

    stage_a(0, buf0)
    fire(buf0)

    def body(j, carry):
        k0 = j * 2
        k1 = k0 + 1
        stage_a(k1, buf1)
        fire(buf1)
        wait_gathers(buf0)

        @pl.when(j > 0)
        def _():
            wait_out(buf0)

        stage_c(k0, buf0)
        fire_out(k0, buf0)

        k2 = jnp.minimum(k0 + 2, NCHUNK - 1)
        stage_a(k2, buf0)
        fire(buf0)
        wait_gathers(buf1)

        @pl.when(j > 0)
        def _():
            wait_out(buf1)

        stage_c(k1, buf1)
        fire_out(k1, buf1)
        return carry

    lax.fori_loop(0, NCHUNK // 2, body, 0)
    wait_gathers(buf0)
    wait_out(buf0)
    wait_out(buf1)


def _buf_scratch():
    return [
        pltpu.VMEM((8 * C,), jnp.int32),
        pltpu.VMEM((4 * C,), jnp.int32),
        pltpu.VMEM((4 * C,), jnp.int32),
        pltpu.VMEM((4 * C,), jnp.int32),
        pltpu.VMEM((8 * C, L), jnp.int32),
        pltpu.VMEM((4 * C, L), jnp.int32),
        pltpu.VMEM((4 * C, L), jnp.int32),
        pltpu.VMEM((4 * C, L), jnp.int32),
        pltpu.VMEM((6, C), jnp.float32),
        pltpu.VMEM((C, F), jnp.float32),
    ]


@jax.jit
def _decomp_grid(grid_t, p0_t, p1_t, p2_t, xs, ys, zs):
    mesh = plsc.VectorSubcoreMesh(core_axis_name="c", subcore_axis_name="s")
    return pl.kernel(
        _sc_body,
        out_type=jax.ShapeDtypeStruct((B, F), jnp.float32),
        mesh=mesh,
        compiler_params=pltpu.CompilerParams(use_tc_tiling_on_sc=False),
        scratch_types=(
            [pltpu.VMEM((SPT,), jnp.float32)] * 3
            + _buf_scratch() + _buf_scratch()
            + [pltpu.SemaphoreType.DMA] * 4
        ),
    )(grid_t, p0_t, p1_t, p2_t, xs, ys, zs)


def _to_table(arr, n, delta):
    t = arr.reshape(F, n)
    if delta:
        t = t - 1.0
    u = lax.bitcast_convert_type(t, jnp.uint32)
    rnd = (u + jnp.uint32(0x7FFF) + ((u >> 16) & 1)) >> 16
    r3 = rnd.reshape(L, 2, n)
    w2 = r3[:, 0, :] | (r3[:, 1, :] << 16)
    return lax.bitcast_convert_type(w2, jnp.int32).T


def kernel(x, feature_grid_3d, plane0, plane1, plane2):
    grid_t = _to_table(feature_grid_3d, GD * GD * GD, False)
    p0_t = _to_table(plane0, PD * PD, True)
    p1_t = _to_table(plane1, PD * PD, True)
    p2_t = _to_table(plane2, PD * PD, True)
    xs = x[:, 0]
    ys = x[:, 1]
    zs = x[:, 2]
    return _decomp_grid(grid_t, p0_t, p1_t, p2_t, xs, ys, zs)


# speedup vs baseline: 2.4346x; 2.4346x over previous
import jax
import jax.numpy as jnp
from jax import lax
from jax.experimental import pallas as pl
from jax.experimental.pallas import tpu as pltpu
from jax.experimental.pallas import tpu_sc as plsc

F = 32
B = 262144
GD = 96
PD = 384
NC, NS, L = 2, 16, 16
NW = NC * NS
SPT = B // NW
C = 64
NCHUNK = SPT // C
NG = C // L

G_OFF = (0, 1, GD, GD + 1, GD * GD, GD * GD + 1, GD * GD + GD, GD * GD + GD + 1)
P_OFF = (0, 1, PD, PD + 1)


def _sc_body(grid_t, p0_t, p1_t, p2_t, xs, ys, zs, out,
             xv, yv, zv,
             idxg0, idxp00, idxp10, idxp20, rg0, rp00, rp10, rp20, fr0, ob0,
             idxg1, idxp01, idxp11, idxp21, rg1, rp01, rp11, rp21, fr1, ob1,
             gsem0, gsem1, osem0, osem1):
    wid = lax.axis_index("s") * NC + lax.axis_index("c")
    tbase = wid * SPT

    buf0 = (idxg0, idxp00, idxp10, idxp20, rg0, rp00, rp10, rp20, fr0, ob0,
            gsem0, osem0)
    buf1 = (idxg1, idxp01, idxp11, idxp21, rg1, rp01, rp11, rp21, fr1, ob1,
            gsem1, osem1)

    pltpu.sync_copy(xs.at[pl.ds(tbase, SPT)], xv)
    pltpu.sync_copy(ys.at[pl.ds(tbase, SPT)], yv)
    pltpu.sync_copy(zs.at[pl.ds(tbase, SPT)], zv)

    lane = lax.iota(jnp.int32, L)
    g_scale = jnp.float32(0.5 * (GD - 1))
    p_scale = jnp.float32(0.5 * (PD - 1))

    def stage_a(kc, buf):
        idxg, idxp0, idxp1, idxp2, _, _, _, _, fr = buf[:9]

        def body(g, kcc):
            s0 = pl.multiple_of(kcc * C + g * L, L)
            g16 = pl.multiple_of(g * L, L)
            cx = xv[pl.ds(s0, L)] + 1.0
            cy = yv[pl.ds(s0, L)] + 1.0
            cz = zv[pl.ds(s0, L)] + 1.0

            def split(t, scale, hi):
                ti = t * scale
                i0 = jnp.minimum(ti.astype(jnp.int32), hi)
                frac = ti - i0.astype(jnp.float32)
                return i0, frac

            x0, fx = split(cx, g_scale, GD - 2)
            y0, fy = split(cy, g_scale, GD - 2)
            z0, fz = split(cz, g_scale, GD - 2)
            xp, fxp = split(cx, p_scale, PD - 2)
            yp, fyp = split(cy, p_scale, PD - 2)
            zp, fzp = split(cz, p_scale, PD - 2)

            r0 = (z0 * GD + y0) * GD + x0
            for c, off in enumerate(G_OFF):
                idxg[pl.ds(c * C + g16, L)] = r0 + off
            rp0 = yp * PD + xp
            rp1 = zp * PD + xp
            rp2 = zp * PD + yp
            for c, off in enumerate(P_OFF):
                idxp0[pl.ds(c * C + g16, L)] = rp0 + off
                idxp1[pl.ds(c * C + g16, L)] = rp1 + off
                idxp2[pl.ds(c * C + g16, L)] = rp2 + off
            for j, v in enumerate((fx, fy, fz, fxp, fyp, fzp)):
                fr[j, pl.ds(g16, L)] = v
            return kcc

        lax.fori_loop(0, NG, body, kc)

    def gather_pairs(buf):
        idxg, idxp0, idxp1, idxp2, rg, rp0, rp1, rp2 = buf[:8]
        gsem = buf[10]
        return ((grid_t.at[idxg], rg, gsem),
                (p0_t.at[idxp0], rp0, gsem),
                (p1_t.at[idxp1], rp1, gsem),
                (p2_t.at[idxp2], rp2, gsem))

    def fire(buf):
        for src, dst, sem in gather_pairs(buf):
            pltpu.async_copy(src, dst, sem)

    def wait_gathers(buf):
        for src, dst, sem in gather_pairs(buf):
            pltpu.make_async_copy(src, dst, sem).wait()

    def stage_c(kc, buf):
        rg, rp0, rp1, rp2, fr, ob = buf[4:10]
        prefs = (rp0, rp1, rp2)

        def body(g, kcc):
            g16 = pl.multiple_of(g * L, L)
            fxv = fr[0, pl.ds(g16, L)]
            fyv = fr[1, pl.ds(g16, L)]
            fzv = fr[2, pl.ds(g16, L)]
            fxpv = fr[3, pl.ds(g16, L)]
            fypv = fr[4, pl.ds(g16, L)]
            fzpv = fr[5, pl.ds(g16, L)]
            for sl in range(L):
                s = g16 + sl
                fx = jnp.full((L,), fxv[sl], jnp.float32)
                fy = jnp.full((L,), fyv[sl], jnp.float32)
                fz = jnp.full((L,), fzv[sl], jnp.float32)
                fxp = jnp.full((L,), fxpv[sl], jnp.float32)
                fyp = jnp.full((L,), fypv[sl], jnp.float32)
                fzp = jnp.full((L,), fzpv[sl], jnp.float32)
                wx0, wx1 = 1.0 - fx, fx
                wy0, wy1 = 1.0 - fy, fy
                wz0, wz1 = 1.0 - fz, fz
                a00 = wz0 * wy0
                a01 = wz0 * wy1
                a10 = wz1 * wy0
                a11 = wz1 * wy1
                wg = (a00 * wx0, a00 * wx1, a01 * wx0, a01 * wx1,
                      a10 * wx0, a10 * wx1, a11 * wx0, a11 * wx1)
                wxp0, wxp1 = 1.0 - fxp, fxp
                wyp0, wyp1 = 1.0 - fyp, fyp
                wzp0, wzp1 = 1.0 - fzp, fzp
                wp = ((wyp0 * wxp0, wyp0 * wxp1, wyp1 * wxp0, wyp1 * wxp1),
                      (wzp0 * wxp0, wzp0 * wxp1, wzp1 * wxp0, wzp1 * wxp1),
                      (wzp0 * wyp0, wzp0 * wyp1, wzp1 * wyp0, wzp1 * wyp1))

                def halves(w):
                    lo = lax.bitcast_convert_type(lax.shift_left(w, 16), jnp.float32)
                    hi = lax.bitcast_convert_type(
                        lax.bitwise_and(w, jnp.int32(-65536)), jnp.float32)
                    return lo, hi

                acc = [None, None]
                for c in range(8):
                    v0, v1 = halves(rg[c * C + s, :])
                    for h, v in enumerate((v0, v1)):
                        t = wg[c] * v
                        acc[h] = t if c == 0 else acc[h] + t
                for p in range(3):
                    pacc = [None, None]
                    for c in range(4):
                        v0, v1 = halves(prefs[p][c * C + s, :])
                        for h, v in enumerate((v0, v1)):
                            t = wp[p][c] * v
                            pacc[h] = t if c == 0 else pacc[h] + t
                    for h in range(2):
                        acc[h] = acc[h] * (1.0 + pacc[h])
                for h in range(2):
                    ob[s, pl.ds(h * L, L)] = acc[h]
            return kcc

        lax.fori_loop(0, NG, body, kc)

    def fire_out(kc, buf):
        ob, _, osem = buf[9:12]
        pltpu.async_copy(ob, out.at[pl.ds(tbase + kc * C, C)], osem)

    def wait_out(buf):
        ob, _, osem = buf[9:12]
        pltpu.make_async_copy(ob, out.at[pl.ds(tbase, C)], osem).wait()

    stage_a(0, buf0)
    fire(buf0)

    def body(j, carry):
        k0 = j * 2
        k1 = k0 + 1
        stage_a(k1, buf1)
        fire(buf1)
        wait_gathers(buf0)

        @pl.when(j > 0)
        def _():
            wait_out(buf0)

        stage_c(k0, buf0)
        fire_out(k0, buf0)

        k2 = jnp.minimum(k0 + 2, NCHUNK - 1)
        stage_a(k2, buf0)
        fire(buf0)
        wait_gathers(buf1)

        @pl.when(j > 0)
        def _():
            wait_out(buf1)

        stage_c(k1, buf1)
        fire_out(k1, buf1)
        return carry

    lax.fori_loop(0, NCHUNK // 2, body, 0)
    wait_gathers(buf0)
    wait_out(buf0)
    wait_out(buf1)


def _buf_scratch():
    return [
        pltpu.VMEM((8 * C,), jnp.int32),
        pltpu.VMEM((4 * C,), jnp.int32),
        pltpu.VMEM((4 * C,), jnp.int32),
        pltpu.VMEM((4 * C,), jnp.int32),
        pltpu.VMEM((8 * C, L), jnp.int32),
        pltpu.VMEM((4 * C, L), jnp.int32),
        pltpu.VMEM((4 * C, L), jnp.int32),
        pltpu.VMEM((4 * C, L), jnp.int32),
        pltpu.VMEM((6, C), jnp.float32),
        pltpu.VMEM((C, F), jnp.float32),
    ]


@jax.jit
def _decomp_grid(grid_t, p0_t, p1_t, p2_t, xs, ys, zs):
    mesh = plsc.VectorSubcoreMesh(core_axis_name="c", subcore_axis_name="s")
    return pl.kernel(
        _sc_body,
        out_type=jax.ShapeDtypeStruct((B, F), jnp.float32),
        mesh=mesh,
        compiler_params=pltpu.CompilerParams(use_tc_tiling_on_sc=False),
        scratch_types=(
            [pltpu.VMEM((SPT,), jnp.float32)] * 3
            + _buf_scratch() + _buf_scratch()
            + [pltpu.SemaphoreType.DMA] * 4
        ),
    )(grid_t, p0_t, p1_t, p2_t, xs, ys, zs)


def _to_table(arr, n, delta):
    t = arr[0]
    if delta:
        t = t - 1.0
    u = lax.bitcast_convert_type(t, jnp.uint32)
    rnd = (u + jnp.uint32(0x7FFF) + ((u >> 16) & 1)) >> 16
    w = rnd[:L] | (rnd[L:] << 16)
    return lax.bitcast_convert_type(w, jnp.int32).reshape(L, n).T


def kernel(x, feature_grid_3d, plane0, plane1, plane2):
    grid_t = _to_table(feature_grid_3d, GD * GD * GD, False)
    p0_t = _to_table(plane0, PD * PD, True)
    p1_t = _to_table(plane1, PD * PD, True)
    p2_t = _to_table(plane2, PD * PD, True)
    xs = x[:, 0]
    ys = x[:, 1]
    zs = x[:, 2]
    return _decomp_grid(grid_t, p0_t, p1_t, p2_t, xs, ys, zs)


# speedup vs baseline: 3.0691x; 1.2607x over previous
import jax
import jax.numpy as jnp
from jax import lax
from jax.experimental import pallas as pl
from jax.experimental.pallas import tpu as pltpu
from jax.experimental.pallas import tpu_sc as plsc

F = 32
B = 262144
GD = 96
PD = 384
NC, NS, L = 2, 16, 16
NW = NC * NS
SPT = B // NW
C = 64
NCHUNK = SPT // C
NG = C // L

G_OFF = (0, 1, GD, GD + 1, GD * GD, GD * GD + 1, GD * GD + GD, GD * GD + GD + 1)
P_OFF = (0, 1, PD, PD + 1)


def _sc_body(grid_t, p0_t, p1_t, p2_t, xs, ys, zs, out,
             xv, yv, zv,
             idxg0, idxp00, idxp10, idxp20, rg0, rp00, rp10, rp20, fr0, ob0,
             idxg1, idxp01, idxp11, idxp21, rg1, rp01, rp11, rp21, fr1, ob1,
             gsem0, gsem1, osem0, osem1):
    wid = lax.axis_index("s") * NC + lax.axis_index("c")
    tbase = wid * SPT

    buf0 = (idxg0, idxp00, idxp10, idxp20, rg0, rp00, rp10, rp20, fr0, ob0,
            gsem0, osem0)
    buf1 = (idxg1, idxp01, idxp11, idxp21, rg1, rp01, rp11, rp21, fr1, ob1,
            gsem1, osem1)

    pltpu.sync_copy(xs.at[pl.ds(tbase, SPT)], xv)
    pltpu.sync_copy(ys.at[pl.ds(tbase, SPT)], yv)
    pltpu.sync_copy(zs.at[pl.ds(tbase, SPT)], zv)

    lane = lax.iota(jnp.int32, L)
    g_scale = jnp.float32(0.5 * (GD - 1))
    p_scale = jnp.float32(0.5 * (PD - 1))

    def stage_a(kc, buf):
        idxg, idxp0, idxp1, idxp2, _, _, _, _, fr = buf[:9]

        def body(g, kcc):
            s0 = pl.multiple_of(kcc * C + g * L, L)
            g16 = pl.multiple_of(g * L, L)
            cx = xv[pl.ds(s0, L)] + 1.0
            cy = yv[pl.ds(s0, L)] + 1.0
            cz = zv[pl.ds(s0, L)] + 1.0

            def split(t, scale, hi):
                ti = t * scale
                i0 = jnp.minimum(ti.astype(jnp.int32), hi)
                frac = ti - i0.astype(jnp.float32)
                return i0, frac

            x0, fx = split(cx, g_scale, GD - 2)
            y0, fy = split(cy, g_scale, GD - 2)
            z0, fz = split(cz, g_scale, GD - 2)
            xp, fxp = split(cx, p_scale, PD - 2)
            yp, fyp = split(cy, p_scale, PD - 2)
            zp, fzp = split(cz, p_scale, PD - 2)

            r0 = (z0 * GD + y0) * GD + x0
            for c, off in enumerate(G_OFF):
                idxg[pl.ds(c * C + g16, L)] = r0 + off
            rp0 = yp * PD + xp
            rp1 = zp * PD + xp
            rp2 = zp * PD + yp
            for c, off in enumerate(P_OFF):
                idxp0[pl.ds(c * C + g16, L)] = rp0 + off
                idxp1[pl.ds(c * C + g16, L)] = rp1 + off
                idxp2[pl.ds(c * C + g16, L)] = rp2 + off
            for j, v in enumerate((fx, fy, fz, fxp, fyp, fzp)):
                fr[j, pl.ds(g16, L)] = v
            return kcc

        lax.fori_loop(0, NG, body, kc)

    def gather_pairs(buf):
        idxg, idxp0, idxp1, idxp2, rg, rp0, rp1, rp2 = buf[:8]
        gsem = buf[10]
        return ((grid_t.at[idxg], rg, gsem),
                (p0_t.at[idxp0], rp0, gsem),
                (p1_t.at[idxp1], rp1, gsem),
                (p2_t.at[idxp2], rp2, gsem))

    def fire(buf):
        for src, dst, sem in gather_pairs(buf):
            pltpu.async_copy(src, dst, sem)

    def wait_gathers(buf):
        for src, dst, sem in gather_pairs(buf):
            pltpu.make_async_copy(src, dst, sem).wait()

    def stage_c(kc, buf):
        rg, rp0, rp1, rp2, fr, ob = buf[4:10]
        prefs = (rp0, rp1, rp2)

        def body(g, kcc):
            g16 = pl.multiple_of(g * L, L)
            fxv = fr[0, pl.ds(g16, L)]
            fyv = fr[1, pl.ds(g16, L)]
            fzv = fr[2, pl.ds(g16, L)]
            fxpv = fr[3, pl.ds(g16, L)]
            fypv = fr[4, pl.ds(g16, L)]
            fzpv = fr[5, pl.ds(g16, L)]
            for sl in range(L):
                s = g16 + sl
                fx = jnp.full((L,), fxv[sl], jnp.float32)
                fy = jnp.full((L,), fyv[sl], jnp.float32)
                fz = jnp.full((L,), fzv[sl], jnp.float32)
                fxp = jnp.full((L,), fxpv[sl], jnp.float32)
                fyp = jnp.full((L,), fypv[sl], jnp.float32)
                fzp = jnp.full((L,), fzpv[sl], jnp.float32)
                wx0, wx1 = 1.0 - fx, fx
                wy0, wy1 = 1.0 - fy, fy
                wz0, wz1 = 1.0 - fz, fz
                a00 = wz0 * wy0
                a01 = wz0 * wy1
                a10 = wz1 * wy0
                a11 = wz1 * wy1
                wg = (a00 * wx0, a00 * wx1, a01 * wx0, a01 * wx1,
                      a10 * wx0, a10 * wx1, a11 * wx0, a11 * wx1)
                wxp0, wxp1 = 1.0 - fxp, fxp
                wyp0, wyp1 = 1.0 - fyp, fyp
                wzp0, wzp1 = 1.0 - fzp, fzp
                wp = ((wyp0 * wxp0, wyp0 * wxp1, wyp1 * wxp0, wyp1 * wxp1),
                      (wzp0 * wxp0, wzp0 * wxp1, wzp1 * wxp0, wzp1 * wxp1),
                      (wzp0 * wyp0, wzp0 * wyp1, wzp1 * wyp0, wzp1 * wyp1))

                acc = [None, None]
                for c in range(8):
                    r = c * C + s
                    for h in range(2):
                        v = rg[r, pl.ds(h * L, L)]
                        t = wg[c] * v
                        acc[h] = t if c == 0 else acc[h] + t
                for p in range(3):
                    pacc = [None, None]
                    for c in range(4):
                        r = c * C + s
                        for h in range(2):
                            v = prefs[p][r, pl.ds(h * L, L)]
                            t = wp[p][c] * v
                            pacc[h] = t if c == 0 else pacc[h] + t
                    for h in range(2):
                        acc[h] = acc[h] * pacc[h]
                for h in range(2):
                    ob[s, pl.ds(h * L, L)] = acc[h]
            return kcc

        lax.fori_loop(0, NG, body, kc)

    def fire_out(kc, buf):
        ob, _, osem = buf[9:12]
        pltpu.async_copy(ob, out.at[pl.ds(tbase + kc * C, C)], osem)

    def wait_out(buf):
        ob, _, osem = buf[9:12]
        pltpu.make_async_copy(ob, out.at[pl.ds(tbase, C)], osem).wait()

    stage_a(0, buf0)
    fire(buf0)

    def body(j, carry):
        k0 = j * 2
        k1 = k0 + 1
        stage_a(k1, buf1)
        fire(buf1)
        wait_gathers(buf0)

        @pl.when(j > 0)
        def _():
            wait_out(buf0)

        stage_c(k0, buf0)
        fire_out(k0, buf0)

        k2 = jnp.minimum(k0 + 2, NCHUNK - 1)
        stage_a(k2, buf0)
        fire(buf0)
        wait_gathers(buf1)

        @pl.when(j > 0)
        def _():
            wait_out(buf1)

        stage_c(k1, buf1)
        fire_out(k1, buf1)
        return carry

    lax.fori_loop(0, NCHUNK // 2, body, 0)
    wait_gathers(buf0)
    wait_out(buf0)
    wait_out(buf1)


def _buf_scratch():
    return [
        pltpu.VMEM((8 * C,), jnp.int32),
        pltpu.VMEM((4 * C,), jnp.int32),
        pltpu.VMEM((4 * C,), jnp.int32),
        pltpu.VMEM((4 * C,), jnp.int32),
        pltpu.VMEM((8 * C, F), jnp.float32),
        pltpu.VMEM((4 * C, F), jnp.float32),
        pltpu.VMEM((4 * C, F), jnp.float32),
        pltpu.VMEM((4 * C, F), jnp.float32),
        pltpu.VMEM((6, C), jnp.float32),
        pltpu.VMEM((C, F), jnp.float32),
    ]


@jax.jit
def _decomp_grid(grid_t, p0_t, p1_t, p2_t, xs, ys, zs):
    mesh = plsc.VectorSubcoreMesh(core_axis_name="c", subcore_axis_name="s")
    return pl.kernel(
        _sc_body,
        out_type=jax.ShapeDtypeStruct((B, F), jnp.float32),
        mesh=mesh,
        compiler_params=pltpu.CompilerParams(use_tc_tiling_on_sc=False),
        scratch_types=(
            [pltpu.VMEM((SPT,), jnp.float32)] * 3
            + _buf_scratch() + _buf_scratch()
            + [pltpu.SemaphoreType.DMA] * 4
        ),
    )(grid_t, p0_t, p1_t, p2_t, xs, ys, zs)


def _to_table(arr, n):
    return arr.reshape(F, n).T


def kernel(x, feature_grid_3d, plane0, plane1, plane2):
    grid_t = _to_table(feature_grid_3d, GD * GD * GD)
    p0_t = _to_table(plane0, PD * PD)
    p1_t = _to_table(plane1, PD * PD)
    p2_t = _to_table(plane2, PD * PD)
    xs = x[:, 0]
    ys = x[:, 1]
    zs = x[:, 2]
    return _decomp_grid(grid_t, p0_t, p1_t, p2_t, xs, ys, zs)


# speedup vs baseline: 3.0711x; 1.0006x over previous
import jax
import jax.numpy as jnp
from jax import lax
from jax.experimental import pallas as pl
from jax.experimental.pallas import tpu as pltpu
from jax.experimental.pallas import tpu_sc as plsc

F = 32
B = 262144
GD = 96
PD = 384
NC, NS, L = 2, 16, 16
NW = NC * NS
SPT = B // NW
C = 64
NCHUNK = SPT // C
NG = C // L

G_OFF = (0, 1, GD, GD + 1, GD * GD, GD * GD + 1, GD * GD + GD, GD * GD + GD + 1)
P_OFF = (0, 1, PD, PD + 1)


def _sc_body(grid_t, p0_t, p1_t, p2_t, xs, ys, zs, out,
             xv, yv, zv,
             idxg0, idxp00, idxp10, idxp20, rg0, rp00, rp10, rp20, fr0, ob0,
             idxg1, idxp01, idxp11, idxp21, rg1, rp01, rp11, rp21, fr1, ob1,
             gsem0, gsem1, osem0, osem1):
    wid = lax.axis_index("s") * NC + lax.axis_index("c")
    tbase = wid * SPT

    buf0 = (idxg0, idxp00, idxp10, idxp20, rg0, rp00, rp10, rp20, fr0, ob0,
            gsem0, osem0)
    buf1 = (idxg1, idxp01, idxp11, idxp21, rg1, rp01, rp11, rp21, fr1, ob1,
            gsem1, osem1)

    pltpu.sync_copy(xs.at[pl.ds(tbase, SPT)], xv)
    pltpu.sync_copy(ys.at[pl.ds(tbase, SPT)], yv)
    pltpu.sync_copy(zs.at[pl.ds(tbase, SPT)], zv)

    lane = lax.iota(jnp.int32, L)
    g_scale = jnp.float32(0.5 * (GD - 1))
    p_scale = jnp.float32(0.5 * (PD - 1))

    def stage_a(kc, buf):
        idxg, idxp0, idxp1, idxp2, _, _, _, _, fr = buf[:9]

        def body(g, kcc):
            s0 = pl.multiple_of(kcc * C + g * L, L)
            g16 = pl.multiple_of(g * L, L)
            cx = xv[pl.ds(s0, L)] + 1.0
            cy = yv[pl.ds(s0, L)] + 1.0
            cz = zv[pl.ds(s0, L)] + 1.0

            def split(t, scale, hi):
                ti = t * scale
                i0 = jnp.minimum(ti.astype(jnp.int32), hi)
                frac = ti - i0.astype(jnp.float32)
                return i0, frac

            x0, fx = split(cx, g_scale, GD - 2)
            y0, fy = split(cy, g_scale, GD - 2)
            z0, fz = split(cz, g_scale, GD - 2)
            xp, fxp = split(cx, p_scale, PD - 2)
            yp, fyp = split(cy, p_scale, PD - 2)
            zp, fzp = split(cz, p_scale, PD - 2)

            r0 = (z0 * GD + y0) * GD + x0
            for c, off in enumerate(G_OFF):
                idxg[pl.ds(c * C + g16, L)] = r0 + off
            rp0 = yp * PD + xp
            rp1 = zp * PD + xp
            rp2 = zp * PD + yp
            for c, off in enumerate(P_OFF):
                idxp0[pl.ds(c * C + g16, L)] = rp0 + off
                idxp1[pl.ds(c * C + g16, L)] = rp1 + off
                idxp2[pl.ds(c * C + g16, L)] = rp2 + off
            for j, v in enumerate((fx, fy, fz, fxp, fyp, fzp)):
                fr[j, pl.ds(g16, L)] = v
            return kcc

        lax.fori_loop(0, NG, body, kc)

    def gather_pairs(buf):
        idxg, idxp0, idxp1, idxp2, rg, rp0, rp1, rp2 = buf[:8]
        gsem = buf[10]
        return ((grid_t.at[idxg], rg, gsem),
                (p0_t.at[idxp0], rp0, gsem),
                (p1_t.at[idxp1], rp1, gsem),
                (p2_t.at[idxp2], rp2, gsem))

    def fire(buf):
        for src, dst, sem in gather_pairs(buf):
            pltpu.async_copy(src, dst, sem)

    def wait_gathers(buf):
        for src, dst, sem in gather_pairs(buf):
            pltpu.make_async_copy(src, dst, sem).wait()

    def stage_c(kc, buf):
        rg, rp0, rp1, rp2, fr, ob = buf[4:10]
        prefs = (rp0, rp1, rp2)

        def body(g, kcc):
            g16 = pl.multiple_of(g * L, L)
            fxv = fr[0, pl.ds(g16, L)]
            fyv = fr[1, pl.ds(g16, L)]
            fzv = fr[2, pl.ds(g16, L)]
            fxpv = fr[3, pl.ds(g16, L)]
            fypv = fr[4, pl.ds(g16, L)]
            fzpv = fr[5, pl.ds(g16, L)]
            for sl in range(L):
                s = g16 + sl
                fx = jnp.full((L,), fxv[sl], jnp.float32)
                fy = jnp.full((L,), fyv[sl], jnp.float32)
                fz = jnp.full((L,), fzv[sl], jnp.float32)
                fxp = jnp.full((L,), fxpv[sl], jnp.float32)
                fyp = jnp.full((L,), fypv[sl], jnp.float32)
                fzp = jnp.full((L,), fzpv[sl], jnp.float32)
                wx0, wx1 = 1.0 - fx, fx
                wy0, wy1 = 1.0 - fy, fy
                wz0, wz1 = 1.0 - fz, fz
                a00 = wz0 * wy0
                a01 = wz0 * wy1
                a10 = wz1 * wy0
                a11 = wz1 * wy1
                wg = (a00 * wx0, a00 * wx1, a01 * wx0, a01 * wx1,
                      a10 * wx0, a10 * wx1, a11 * wx0, a11 * wx1)
                wxp0, wxp1 = 1.0 - fxp, fxp
                wyp0, wyp1 = 1.0 - fyp, fyp
                wzp0, wzp1 = 1.0 - fzp, fzp
                wp = ((wyp0 * wxp0, wyp0 * wxp1, wyp1 * wxp0, wyp1 * wxp1),
                      (wzp0 * wxp0, wzp0 * wxp1, wzp1 * wxp0, wzp1 * wxp1),
                      (wzp0 * wyp0, wzp0 * wyp1, wzp1 * wyp0, wzp1 * wyp1))

                acc = [None, None]
                for c in range(8):
                    r = c * C + s
                    for h in range(2):
                        v = rg[r, pl.ds(h * L, L)]
                        t = wg[c] * v
                        acc[h] = t if c == 0 else acc[h] + t
                for p in range(3):
                    pacc = [None, None]
                    for c in range(4):
                        r = c * C + s
                        for h in range(2):
                            v = prefs[p][r, pl.ds(h * L, L)]
                            t = wp[p][c] * v
                            pacc[h] = t if c == 0 else pacc[h] + t
                    for h in range(2):
                        acc[h] = acc[h] * pacc[h]
                for h in range(2):
                    ob[s, pl.ds(h * L, L)] = acc[h]
            return kcc

        lax.fori_loop(0, NG, body, kc)

    def fire_out(kc, buf):
        ob, _, osem = buf[9:12]
        pltpu.async_copy(ob, out.at[pl.ds(tbase + kc * C, C)], osem)

    def wait_out(buf):
        ob, _, osem = buf[9:12]
        pltpu.make_async_copy(ob, out.at[pl.ds(tbase, C)], osem).wait()

    stage_a(0, buf0)
    fire(buf0)

    def body(j, carry):
        k0 = j * 2
        k1 = k0 + 1
        stage_a(k1, buf1)
        fire(buf1)
        wait_gathers(buf0)

        @pl.when(j > 0)
        def _():
            wait_out(buf0)

        stage_c(k0, buf0)
        fire_out(k0, buf0)

        k2 = jnp.minimum(k0 + 2, NCHUNK - 1)
        stage_a(k2, buf0)
        fire(buf0)
        wait_gathers(buf1)

        @pl.when(j > 0)
        def _():
            wait_out(buf1)

        stage_c(k1, buf1)
        fire_out(k1, buf1)
        return carry

    lax.fori_loop(0, NCHUNK // 2, body, 0)
    wait_gathers(buf0)
    wait_out(buf0)
    wait_out(buf1)


def _buf_scratch():
    return [
        pltpu.VMEM((8 * C,), jnp.int32),
        pltpu.VMEM((4 * C,), jnp.int32),
        pltpu.VMEM((4 * C,), jnp.int32),
        pltpu.VMEM((4 * C,), jnp.int32),
        pltpu.VMEM((8 * C, F), jnp.float32),
        pltpu.VMEM((4 * C, F), jnp.float32),
        pltpu.VMEM((4 * C, F), jnp.float32),
        pltpu.VMEM((4 * C, F), jnp.float32),
        pltpu.VMEM((6, C), jnp.float32),
        pltpu.VMEM((C, F), jnp.float32),
    ]


@jax.jit
def _decomp_grid(grid_t, p0_t, p1_t, p2_t, xs, ys, zs):
    mesh = plsc.VectorSubcoreMesh(core_axis_name="c", subcore_axis_name="s")
    return pl.kernel(
        _sc_body,
        out_type=jax.ShapeDtypeStruct((B, F), jnp.float32),
        mesh=mesh,
        compiler_params=pltpu.CompilerParams(use_tc_tiling_on_sc=False),
        scratch_types=(
            [pltpu.VMEM((SPT,), jnp.float32)] * 3
            + _buf_scratch() + _buf_scratch()
            + [pltpu.SemaphoreType.DMA] * 4
        ),
    )(grid_t, p0_t, p1_t, p2_t, xs, ys, zs)


def _to_table(arr, n):
    return arr.reshape(F, n).T


def kernel(x, feature_grid_3d, plane0, plane1, plane2):
    grid_t = _to_table(feature_grid_3d, GD * GD * GD)
    p0_t = _to_table(plane0, PD * PD)
    p1_t = _to_table(plane1, PD * PD)
    p2_t = _to_table(plane2, PD * PD)
    xt = x.T
    return _decomp_grid(grid_t, p0_t, p1_t, p2_t, xt[0], xt[1], xt[2])


# speedup vs baseline: 3.0716x; 1.0002x over previous
import jax
import jax.numpy as jnp
from jax import lax
from jax.experimental import pallas as pl
from jax.experimental.pallas import tpu as pltpu
from jax.experimental.pallas import tpu_sc as plsc

F = 32
B = 262144
GD = 96
PD = 384
NC, NS, L = 2, 16, 16
NW = NC * NS
SPT = B // NW
C = 64
NCHUNK = SPT // C
NG = C // L

G_OFF = (0, 1, GD, GD + 1, GD * GD, GD * GD + 1, GD * GD + GD, GD * GD + GD + 1)
P_OFF = (0, 1, PD, PD + 1)


def _sc_body(grid_t, p0_t, p1_t, p2_t, xs, ys, zs, out,
             xv, yv, zv,
             idxg0, idxp00, idxp10, idxp20, rg0, rp00, rp10, rp20, fr0, ob0,
             idxg1, idxp01, idxp11, idxp21, rg1, rp01, rp11, rp21, fr1, ob1,
             gsem0, gsem1, osem0, osem1):
    wid = lax.axis_index("s") * NC + lax.axis_index("c")
    tbase = wid * SPT

    buf0 = (idxg0, idxp00, idxp10, idxp20, rg0, rp00, rp10, rp20, fr0, ob0,
            gsem0, osem0)
    buf1 = (idxg1, idxp01, idxp11, idxp21, rg1, rp01, rp11, rp21, fr1, ob1,
            gsem1, osem1)

    pltpu.sync_copy(xs.at[pl.ds(tbase, SPT)], xv)
    pltpu.sync_copy(ys.at[pl.ds(tbase, SPT)], yv)
    pltpu.sync_copy(zs.at[pl.ds(tbase, SPT)], zv)

    g_scale = jnp.float32(0.5 * (GD - 1))
    p_scale = jnp.float32(0.5 * (PD - 1))

    def stage_a(kc, buf):
        idxg, idxp0, idxp1, idxp2, _, _, _, _, fr = buf[:9]

        def body(g, kcc):
            s0 = pl.multiple_of(kcc * C + g * L, L)
            g16 = pl.multiple_of(g * L, L)
            cx = xv[pl.ds(s0, L)] + 1.0
            cy = yv[pl.ds(s0, L)] + 1.0
            cz = zv[pl.ds(s0, L)] + 1.0

            def split(t, scale, hi):
                ti = t * scale
                i0 = jnp.minimum(ti.astype(jnp.int32), hi)
                frac = ti - i0.astype(jnp.float32)
                return i0, frac

            x0, fx = split(cx, g_scale, GD - 2)
            y0, fy = split(cy, g_scale, GD - 2)
            z0, fz = split(cz, g_scale, GD - 2)
            xp, fxp = split(cx, p_scale, PD - 2)
            yp, fyp = split(cy, p_scale, PD - 2)
            zp, fzp = split(cz, p_scale, PD - 2)

            r0 = (z0 * GD + y0) * GD + x0
            for c, off in enumerate(G_OFF):
                idxg[pl.ds(c * C + g16, L)] = r0 + off
            rp0 = yp * PD + xp
            rp1 = zp * PD + xp
            rp2 = zp * PD + yp
            for c, off in enumerate(P_OFF):
                idxp0[pl.ds(c * C + g16, L)] = rp0 + off
                idxp1[pl.ds(c * C + g16, L)] = rp1 + off
                idxp2[pl.ds(c * C + g16, L)] = rp2 + off
            for j, v in enumerate((fx, fy, fz, fxp, fyp, fzp)):
                fr[j, pl.ds(g16, L)] = v
            return kcc

        lax.fori_loop(0, NG, body, kc)

    def gather_pairs(buf):
        idxg, idxp0, idxp1, idxp2, rg, rp0, rp1, rp2 = buf[:8]
        gsem = buf[10]
        return ((grid_t.at[idxg], rg, gsem),
                (p0_t.at[idxp0], rp0, gsem),
                (p1_t.at[idxp1], rp1, gsem),
                (p2_t.at[idxp2], rp2, gsem))

    def fire(buf):
        for src, dst, sem in gather_pairs(buf):
            pltpu.async_copy(src, dst, sem)

    def wait_gathers(buf):
        for src, dst, sem in gather_pairs(buf):
            pltpu.make_async_copy(src, dst, sem).wait()

    def stage_c(kc, buf):
        rg, rp0, rp1, rp2, fr, ob = buf[4:10]
        prefs = (rp0, rp1, rp2)

        def body(g, kcc):
            g16 = pl.multiple_of(g * L, L)
            fxv = fr[0, pl.ds(g16, L)]
            fyv = fr[1, pl.ds(g16, L)]
            fzv = fr[2, pl.ds(g16, L)]
            fxpv = fr[3, pl.ds(g16, L)]
            fypv = fr[4, pl.ds(g16, L)]
            fzpv = fr[5, pl.ds(g16, L)]
            for sl in range(L):
                s = g16 + sl
                fx = jnp.full((L,), fxv[sl], jnp.float32)
                fy = jnp.full((L,), fyv[sl], jnp.float32)
                fz = jnp.full((L,), fzv[sl], jnp.float32)
                fxp = jnp.full((L,), fxpv[sl], jnp.float32)
                fyp = jnp.full((L,), fypv[sl], jnp.float32)
                fzp = jnp.full((L,), fzpv[sl], jnp.float32)
                wx0, wx1 = 1.0 - fx, fx
                wy0, wy1 = 1.0 - fy, fy
                wz0, wz1 = 1.0 - fz, fz
                a00 = wz0 * wy0
                a01 = wz0 * wy1
                a10 = wz1 * wy0
                a11 = wz1 * wy1
                wg = (a00 * wx0, a00 * wx1, a01 * wx0, a01 * wx1,
                      a10 * wx0, a10 * wx1, a11 * wx0, a11 * wx1)
                wxp0, wxp1 = 1.0 - fxp, fxp
                wyp0, wyp1 = 1.0 - fyp, fyp
                wzp0, wzp1 = 1.0 - fzp, fzp
                wp = ((wyp0 * wxp0, wyp0 * wxp1, wyp1 * wxp0, wyp1 * wxp1),
                      (wzp0 * wxp0, wzp0 * wxp1, wzp1 * wxp0, wzp1 * wxp1),
                      (wzp0 * wyp0, wzp0 * wyp1, wzp1 * wyp0, wzp1 * wyp1))

                acc = [None, None]
                for c in range(8):
                    r = c * C + s
                    for h in range(2):
                        v = rg[r, pl.ds(h * L, L)]
                        t = wg[c] * v
                        acc[h] = t if c == 0 else acc[h] + t
                for p in range(3):
                    pacc = [None, None]
                    for c in range(4):
                        r = c * C + s
                        for h in range(2):
                            v = prefs[p][r, pl.ds(h * L, L)]
                            t = wp[p][c] * v
                            pacc[h] = t if c == 0 else pacc[h] + t
                    for h in range(2):
                        acc[h] = acc[h] * pacc[h]
                for h in range(2):
                    ob[s, pl.ds(h * L, L)] = acc[h]
            return kcc

        lax.fori_loop(0, NG, body, kc)

    def fire_out(kc, buf):
        ob, _, osem = buf[9:12]
        pltpu.async_copy(ob, out.at[pl.ds(tbase + kc * C, C)], osem)

    def wait_out(buf):
        ob, _, osem = buf[9:12]
        pltpu.make_async_copy(ob, out.at[pl.ds(tbase, C)], osem).wait()

    stage_a(0, buf0)
    fire(buf0)

    def body(j, carry):
        k0 = j * 2
        k1 = k0 + 1
        stage_a(k1, buf1)
        fire(buf1)
        wait_gathers(buf0)

        @pl.when(j > 0)
        def _():
            wait_out(buf0)

        stage_c(k0, buf0)
        fire_out(k0, buf0)

        k2 = jnp.minimum(k0 + 2, NCHUNK - 1)
        stage_a(k2, buf0)
        fire(buf0)
        wait_gathers(buf1)

        @pl.when(j > 0)
        def _():
            wait_out(buf1)

        stage_c(k1, buf1)
        fire_out(k1, buf1)
        return carry

    lax.fori_loop(0, NCHUNK // 2, body, 0)
    wait_gathers(buf0)
    wait_out(buf0)
    wait_out(buf1)


def _buf_scratch():
    return [
        pltpu.VMEM((8 * C,), jnp.int32),
        pltpu.VMEM((4 * C,), jnp.int32),
        pltpu.VMEM((4 * C,), jnp.int32),
        pltpu.VMEM((4 * C,), jnp.int32),
        pltpu.VMEM((8 * C, F), jnp.float32),
        pltpu.VMEM((4 * C, F), jnp.float32),
        pltpu.VMEM((4 * C, F), jnp.float32),
        pltpu.VMEM((4 * C, F), jnp.float32),
        pltpu.VMEM((6, C), jnp.float32),
        pltpu.VMEM((C, F), jnp.float32),
    ]


@jax.jit
def _decomp_grid(grid_t, p0_t, p1_t, p2_t, xs, ys, zs):
    mesh = plsc.VectorSubcoreMesh(core_axis_name="c", subcore_axis_name="s")
    return pl.kernel(
        _sc_body,
        out_type=jax.ShapeDtypeStruct((B, F), jnp.float32),
        mesh=mesh,
        compiler_params=pltpu.CompilerParams(use_tc_tiling_on_sc=False),
        scratch_types=(
            [pltpu.VMEM((SPT,), jnp.float32)] * 3
            + _buf_scratch() + _buf_scratch()
            + [pltpu.SemaphoreType.DMA] * 4
        ),
    )(grid_t, p0_t, p1_t, p2_t, xs, ys, zs)


def _to_table(arr, n):
    return jnp.moveaxis(arr[0], 0, -1).reshape(n, F)


def kernel(x, feature_grid_3d, plane0, plane1, plane2):
    grid_t = _to_table(feature_grid_3d, GD * GD * GD)
    p0_t = _to_table(plane0, PD * PD)
    p1_t = _to_table(plane1, PD * PD)
    p2_t = _to_table(plane2, PD * PD)
    xt = x.T
    return _decomp_grid(grid_t, p0_t, p1_t, p2_t, xt[0], xt[1], xt[2])
